# baseline (speedup 1.0000x reference)
import functools

import jax
import jax.numpy as jnp
from jax.experimental import pallas as pl
from jax.experimental.pallas import tpu as pltpu
from jax.experimental.pallas import tpu_sc as plsc

_B, _L, _K = 2, 1024, 20
_DN, _DE, _DIDX = 128, 32, 32
_DND = _DN + _DIDX
_DOUT = 20
_BL = _B * _L
_N = _BL * _K
_EPS = 1e-5
_T = 256
_GW = 128

_f32 = jnp.float32


def _full_spec(a):
    nd = a.ndim
    return pl.BlockSpec(a.shape, lambda i, _nd=nd: (0,) * _nd)


def _stats(s, ss, c):
    shape = s.shape
    n = s.size
    if n % 128 == 0:
        s = s.reshape(n // 128, 128)
        ss = ss.reshape(n // 128, 128)
    mu = s * c
    var = jnp.maximum(ss * c - mu * mu, 0.0)
    inv = jax.lax.rsqrt(var + _EPS)
    return mu.reshape(shape), inv.reshape(shape)


def _ln(x, g, b):
    c = 1.0 / x.shape[-1]
    s = jnp.sum(x, axis=-1, keepdims=True)
    ss = jnp.sum(x * x, axis=-1, keepdims=True)
    mu, inv = _stats(s, ss, c)
    return (x - mu) * inv * g + b


def _dot(a, b):
    return jnp.dot(a, b, preferred_element_type=_f32)


def _rb(x, ps):
    g1, b1, W1, bb1, g2, b2, W2, bb2 = ps[:8]
    h = jax.nn.relu(_ln(x, g1, b1))
    h = _dot(h, W1) + bb1
    h = jax.nn.relu(_ln(h, g2, b2))
    h = _dot(h, W2) + bb2
    sc = x if len(ps) == 8 else _dot(x, ps[8]) + ps[9]
    return h + sc


def _fl(p):
    return [p['g'][None, :], p['b'][None, :]]


def _frb(p):
    out = _fl(p['ln1']) + [p['fc1']['W'], p['fc1']['b'][None, :]]
    out += _fl(p['ln2']) + [p['fc2']['W'], p['fc2']['b'][None, :]]
    if 'sc' in p:
        out += [p['sc']['W'], p['sc']['b'][None, :]]
    return out


def _flat_rgc(blk, dn):
    rb1 = blk['edge_blocks'][0]
    g1, b1 = rb1['ln1']['g'], rb1['ln1']['b']
    W1 = rb1['fc1']['W']
    Wsc = rb1['sc']['W']
    out = [g1[None, :dn], b1[None, :dn],
           g1[None, dn:2 * dn], b1[None, dn:2 * dn],
           g1[None, 2 * dn:], b1[None, 2 * dn:],
           W1[:dn], W1[dn:2 * dn], W1[2 * dn:], rb1['fc1']['b'][None, :]]
    out += _fl(rb1['ln2']) + [rb1['fc2']['W'], rb1['fc2']['b'][None, :]]
    out += [Wsc[:dn], Wsc[dn:2 * dn], Wsc[2 * dn:], rb1['sc']['b'][None, :]]
    out += _frb(blk['edge_blocks'][1])
    out += _fl(blk['edge_ln'])
    out += _frb(blk['node_blocks'][0])
    out += _frb(blk['node_blocks'][1])
    out += _fl(blk['node_ln'])
    return out


_N_RGC = 48


def _sc_gather(table, idx):
    n = idx.shape[1]
    d = table.shape[1]
    mesh = plsc.VectorSubcoreMesh(core_axis_name="core",
                                  subcore_axis_name="subcore")

    @functools.partial(
        pl.kernel,
        out_type=jax.ShapeDtypeStruct((n, d), table.dtype),
        mesh=mesh)
    def gather_kernel(x_hbm, i_hbm, o_hbm):
        def body(i_vmem, o_vmem):
            pltpu.sync_copy(x_hbm.at[i_vmem.at[0]], o_vmem)

        pltpu.emit_pipeline(
            body,
            grid=(n // _GW,),
            in_specs=[pl.BlockSpec((1, _GW), index_map=lambda i: (0, i))],
            out_specs=[pl.BlockSpec((_GW, d), index_map=lambda i: (i, 0))],
            core_axis_name=("core", "subcore"),
            dimension_semantics=(pltpu.PARALLEL,),
        )(i_hbm, o_hbm)

    return gather_kernel(table, idx)


def _shift_cat(x):
    cols = []
    for o in range(-4, 5):
        if o == 0:
            s = x
        elif o > 0:
            s = jnp.concatenate(
                [x[o:], jnp.zeros((o, x.shape[1]), x.dtype)], axis=0)
        else:
            s = jnp.concatenate(
                [jnp.zeros((-o, x.shape[1]), x.dtype), x[:o]], axis=0)
        cols.append(s)
    return jnp.concatenate(cols, axis=1)


def _feat_body(*refs):
    node_ref, resid_ref, em_ref = refs[:3]
    it = iter(refs[3:])
    take = lambda k: [next(it) for _ in range(k)]
    Wc, bc = take(2)
    nf1, nf2 = take(8), take(8)
    nfg, nfb = take(2)
    (emb,) = take(1)
    idx1 = take(8)
    idxg, idxb = take(2)
    ef1, ef2 = take(10), take(8)
    efg, efb = take(2)
    node_out, idx_out, etab_out = take(3)

    val = lambda rs: [r[...] for r in rs]

    x = node_ref[0]
    h = _dot(_shift_cat(x), Wc[...]) + bc[...]
    h = _rb(h, val(nf1))
    h = _rb(h, val(nf2))
    node_out[0] = jax.nn.relu(_ln(h, nfg[...], nfb[...]))

    rf = resid_ref[0]
    iot = jax.lax.broadcasted_iota(jnp.int32, (_L, 32), 1).astype(_f32)
    oh = (rf == iot).astype(_f32)
    tab = _rb(emb[...], val(idx1))
    tab = jax.nn.relu(_ln(tab, idxg[...], idxb[...]))
    idx_out[0] = _dot(oh, tab)

    e = _rb(em_ref[...], val(ef1))
    e = _rb(e, val(ef2))
    e = jax.nn.relu(_ln(e, efg[...], efb[...]))
    etab_out[...] = jnp.concatenate(
        [e, jnp.zeros((_L, 128 - _DE), _f32)], axis=1)


def _feat_call(node3, residf, edgemat, pa):
    in_specs = [
        pl.BlockSpec((1, _L, 6), lambda i: (i, 0, 0)),
        pl.BlockSpec((1, _L, 1), lambda i: (i, 0, 0)),
        _full_spec(edgemat),
    ] + [_full_spec(a) for a in pa]
    out_shape = [
        jax.ShapeDtypeStruct((_B, _L, _DN), _f32),
        jax.ShapeDtypeStruct((_B, _L, _DIDX), _f32),
        jax.ShapeDtypeStruct((_L, 128), _f32),
    ]
    out_specs = [
        pl.BlockSpec((1, _L, _DN), lambda i: (i, 0, 0)),
        pl.BlockSpec((1, _L, _DIDX), lambda i: (i, 0, 0)),
        pl.BlockSpec((_L, 128), lambda i: (0, 0)),
    ]
    return pl.pallas_call(
        _feat_body, grid=(_B,), in_specs=in_specs, out_specs=out_specs,
        out_shape=out_shape)(node3, residf, edgemat, *pa)


def _rgc_body(dn, dpad, final, *refs):
    node_ref, nbr_ref, edge_ref = refs[:3]
    npar = _N_RGC + (2 if final else 0)
    prefs = refs[3:3 + npar]
    outs = refs[3 + npar:]
    v = [r[...] for r in prefs]
    (eg1i, eb1i, eg1n, eb1n, eg1e, eb1e, W1i, W1n, W1e, bb1,
     eg2, eb2, W2, bb2, Wsci, Wscn, Wsce, bsc) = v[:18]
    rb2 = v[18:26]
    elng, elnb = v[26:28]
    n1 = v[28:38]
    n2 = v[38:46]
    nlng, nlnb = v[46:48]

    dcat = 2 * dn + _DE
    node = node_ref[...][:, :dn]
    nbr3 = nbr_ref[...][:, :, :dn]
    edge3 = edge_ref[...]
    nbr = nbr3.reshape(_K * _T, dn)
    edge = edge3.reshape(_K * _T, _DE)

    sc3 = (_dot(node, Wsci)[None] + _dot(nbr, Wscn).reshape(_K, _T, _DE)
           + _dot(edge, Wsce).reshape(_K, _T, _DE) + bsc[None])

    c = 1.0 / dcat
    s_i = jnp.sum(node, axis=1, keepdims=True)[None]
    ss_i = jnp.sum(node * node, axis=1, keepdims=True)[None]
    s = (s_i + jnp.sum(nbr3, axis=2, keepdims=True)
         + jnp.sum(edge3, axis=2, keepdims=True))
    ss = (ss_i + jnp.sum(nbr3 * nbr3, axis=2, keepdims=True)
          + jnp.sum(edge3 * edge3, axis=2, keepdims=True))
    mu, inv = _stats(s, ss, c)

    a_i = jax.nn.relu((node[None] - mu) * inv * eg1i[None] + eb1i[None])
    a_n = jax.nn.relu((nbr3 - mu) * inv * eg1n[None] + eb1n[None])
    a_e = jax.nn.relu((edge3 - mu) * inv * eg1e[None] + eb1e[None])
    h = (_dot(a_i.reshape(_K * _T, dn), W1i)
         + _dot(a_n.reshape(_K * _T, dn), W1n)
         + _dot(a_e.reshape(_K * _T, _DE), W1e) + bb1)
    h = jax.nn.relu(_ln(h, eg2, eb2))
    h = _dot(h, W2) + bb2
    r1 = h.reshape(_K, _T, _DE) + sc3
    r2 = _rb(r1.reshape(_K * _T, _DE), rb2)
    ef = jax.nn.relu(_ln(r2, elng, elnb))
    edge_out = edge3 + ef.reshape(_K, _T, _DE)

    m = jnp.sum(edge_out, axis=0) * (1.0 / _K)

    hn = jnp.concatenate([node, m], axis=1)
    hh = _rb(hn, n1)
    hh = _rb(hh, n2)
    node_out = node + jax.nn.relu(_ln(hh, nlng, nlnb))

    if final:
        Wout, bout = v[48:50]
        outs[0][...] = _dot(node_out, Wout) + bout
    else:
        if dpad > dn:
            node_out = jnp.concatenate(
                [node_out, jnp.zeros((_T, dpad - dn), _f32)], axis=1)
        outs[0][...] = node_out
        outs[1][...] = edge_out


def _rgc_call(node, nbr3, edge3, pv, dn, final, dpad=None):
    dpad = dn if dpad is None else dpad
    grid = (_BL // _T,)
    in_specs = [
        pl.BlockSpec((_T, dpad), lambda i: (i, 0)),
        pl.BlockSpec((_K, _T, dpad), lambda i: (0, i, 0)),
        pl.BlockSpec((_K, _T, _DE), lambda i: (0, i, 0)),
    ] + [_full_spec(a) for a in pv]
    if final:
        out_shape = [jax.ShapeDtypeStruct((_BL, _DOUT), _f32)]
        out_specs = [pl.BlockSpec((_T, _DOUT), lambda i: (i, 0))]
    else:
        out_shape = [jax.ShapeDtypeStruct((_BL, dpad), _f32),
                     jax.ShapeDtypeStruct((_K, _BL, _DE), _f32)]
        out_specs = [pl.BlockSpec((_T, dpad), lambda i: (i, 0)),
                     pl.BlockSpec((_K, _T, _DE), lambda i: (0, i, 0))]
    return pl.pallas_call(
        functools.partial(_rgc_body, dn, dpad, final),
        grid=grid, in_specs=in_specs, out_specs=out_specs,
        out_shape=out_shape)(node, nbr3, edge3, *pv)


def kernel(node_in, edgemat_in, adjmat_in, masked_resid, params):
    p = params

    adj = adjmat_in.astype(jnp.int32)
    off = (jnp.arange(_B, dtype=jnp.int32) * _L)[:, None, None]
    idxT = jnp.transpose(adj + off, (2, 0, 1)).reshape(1, _N)
    idxE = jnp.transpose(adj, (2, 0, 1)).reshape(1, _N)

    residf = masked_resid.astype(_f32).reshape(_B, _L, 1)
    emb_pad = jnp.zeros((32, _DIDX), _f32).at[:21].set(p['idx_emb'])

    pa = [p['conv0']['W'].reshape(9 * 6, _DN), p['conv0']['b'][None, :]]
    pa += _frb(p['nf_blocks'][0]) + _frb(p['nf_blocks'][1])
    pa += _fl(p['nf_ln'])
    pa += [emb_pad] + _frb(p['idx_blocks'][0]) + _fl(p['idx_ln'])
    pa += _frb(p['ef_blocks'][0]) + _frb(p['ef_blocks'][1])
    pa += _fl(p['ef_ln'])

    node0, idxf, etab = _feat_call(node_in, residf, edgemat_in, pa)

    edge = _sc_gather(etab, idxE).reshape(_K, _BL, 128)[:, :, :_DE]
    node = node0.reshape(_BL, _DN)

    for blk in p['encoder']:
        nbr = _sc_gather(node, idxT).reshape(_K, _BL, _DN)
        node, edge = _rgc_call(node, nbr, edge, _flat_rgc(blk, _DN),
                               _DN, final=False)

    _DP = 256
    node = jnp.concatenate(
        [node, idxf.reshape(_BL, _DIDX),
         jnp.zeros((_BL, _DP - _DND), _f32)], axis=1)

    for bi, blk in enumerate(p['decoder']):
        nbr = _sc_gather(node, idxT).reshape(_K, _BL, _DP)
        pv = _flat_rgc(blk, _DND)
        if bi < len(p['decoder']) - 1:
            node, edge = _rgc_call(node, nbr, edge, pv, _DND,
                                   final=False, dpad=_DP)
        else:
            pv += [p['out']['W'], p['out']['b'][None, :]]
            out = _rgc_call(node, nbr, edge, pv, _DND,
                            final=True, dpad=_DP)[0]

    return out.reshape(_B, _L, _DOUT)


# speedup vs baseline: 1.0173x; 1.0173x over previous
import functools

import jax
import jax.numpy as jnp
from jax.experimental import pallas as pl
from jax.experimental.pallas import tpu as pltpu
from jax.experimental.pallas import tpu_sc as plsc

_B, _L, _K = 2, 1024, 20
_DN, _DE, _DIDX = 128, 32, 32
_DND = _DN + _DIDX
_DOUT = 20
_BL = _B * _L
_N = _BL * _K
_EPS = 1e-5
_T = 256
_GW = 128

_f32 = jnp.float32


def _full_spec(a):
    nd = a.ndim
    return pl.BlockSpec(a.shape, lambda i, _nd=nd: (0,) * _nd)


def _stats(s, ss, c):
    shape = s.shape
    n = s.size
    if n % 128 == 0:
        s = s.reshape(n // 128, 128)
        ss = ss.reshape(n // 128, 128)
    mu = s * c
    var = jnp.maximum(ss * c - mu * mu, 0.0)
    inv = jax.lax.rsqrt(var + _EPS)
    return mu.reshape(shape), inv.reshape(shape)


def _ln(x, g, b):
    c = 1.0 / x.shape[-1]
    s = jnp.sum(x, axis=-1, keepdims=True)
    ss = jnp.sum(x * x, axis=-1, keepdims=True)
    mu, inv = _stats(s, ss, c)
    return (x - mu) * inv * g + b


def _dot(a, b):
    return jnp.dot(a, b, preferred_element_type=_f32)


def _rb(x, ps):
    g1, b1, W1, bb1, g2, b2, W2, bb2 = ps[:8]
    h = jax.nn.relu(_ln(x, g1, b1))
    h = _dot(h, W1) + bb1
    h = jax.nn.relu(_ln(h, g2, b2))
    h = _dot(h, W2) + bb2
    sc = x if len(ps) == 8 else _dot(x, ps[8]) + ps[9]
    return h + sc


def _fl(p):
    return [p['g'][None, :], p['b'][None, :]]


def _frb(p):
    out = _fl(p['ln1']) + [p['fc1']['W'], p['fc1']['b'][None, :]]
    out += _fl(p['ln2']) + [p['fc2']['W'], p['fc2']['b'][None, :]]
    if 'sc' in p:
        out += [p['sc']['W'], p['sc']['b'][None, :]]
    return out


def _flat_rgc(blk, dn):
    rb1 = blk['edge_blocks'][0]
    g1, b1 = rb1['ln1']['g'], rb1['ln1']['b']
    W1 = rb1['fc1']['W']
    Wsc = rb1['sc']['W']
    out = [g1[None, :dn], b1[None, :dn],
           g1[None, dn:2 * dn], b1[None, dn:2 * dn],
           g1[None, 2 * dn:], b1[None, 2 * dn:],
           W1[:dn], W1[dn:2 * dn], W1[2 * dn:], rb1['fc1']['b'][None, :]]
    out += _fl(rb1['ln2']) + [rb1['fc2']['W'], rb1['fc2']['b'][None, :]]
    out += [Wsc[:dn], Wsc[dn:2 * dn], Wsc[2 * dn:], rb1['sc']['b'][None, :]]
    out += _frb(blk['edge_blocks'][1])
    out += _fl(blk['edge_ln'])
    out += _frb(blk['node_blocks'][0])
    out += _frb(blk['node_blocks'][1])
    out += _fl(blk['node_ln'])
    return out


_N_RGC = 48


def _sc_gather(table, idx):
    n = idx.shape[1]
    d = table.shape[1]
    mesh = plsc.VectorSubcoreMesh(core_axis_name="core",
                                  subcore_axis_name="subcore")

    @functools.partial(
        pl.kernel,
        out_type=jax.ShapeDtypeStruct((n, d), table.dtype),
        mesh=mesh)
    def gather_kernel(x_hbm, i_hbm, o_hbm):
        def body(i_vmem, o_vmem):
            pltpu.sync_copy(x_hbm.at[i_vmem.at[0]], o_vmem)

        pltpu.emit_pipeline(
            body,
            grid=(n // _GW,),
            in_specs=[pl.BlockSpec((1, _GW), index_map=lambda i: (0, i))],
            out_specs=[pl.BlockSpec((_GW, d), index_map=lambda i: (i, 0))],
            core_axis_name=("core", "subcore"),
            dimension_semantics=(pltpu.PARALLEL,),
        )(i_hbm, o_hbm)

    return gather_kernel(table, idx)


def _shift_cat(x):
    cols = []
    for o in range(-4, 5):
        if o == 0:
            s = x
        elif o > 0:
            s = jnp.concatenate(
                [x[o:], jnp.zeros((o, x.shape[1]), x.dtype)], axis=0)
        else:
            s = jnp.concatenate(
                [jnp.zeros((-o, x.shape[1]), x.dtype), x[:o]], axis=0)
        cols.append(s)
    return jnp.concatenate(cols, axis=1)


def _feat_body(*refs):
    node_ref, resid_ref, em_ref = refs[:3]
    it = iter(refs[3:])
    take = lambda k: [next(it) for _ in range(k)]
    Wc, bc = take(2)
    nf1, nf2 = take(8), take(8)
    nfg, nfb = take(2)
    (emb,) = take(1)
    idx1 = take(8)
    idxg, idxb = take(2)
    ef1, ef2 = take(10), take(8)
    efg, efb = take(2)
    node_out, idx_out, etab_out = take(3)

    val = lambda rs: [r[...] for r in rs]

    x = node_ref[0]
    h = _dot(_shift_cat(x), Wc[...]) + bc[...]
    h = _rb(h, val(nf1))
    h = _rb(h, val(nf2))
    node_out[0] = jax.nn.relu(_ln(h, nfg[...], nfb[...]))

    rf = resid_ref[0]
    iot = jax.lax.broadcasted_iota(jnp.int32, (_L, 32), 1).astype(_f32)
    oh = (rf == iot).astype(_f32)
    tab = _rb(emb[...], val(idx1))
    tab = jax.nn.relu(_ln(tab, idxg[...], idxb[...]))
    idx_out[0] = _dot(oh, tab)

    e = _rb(em_ref[...], val(ef1))
    e = _rb(e, val(ef2))
    e = jax.nn.relu(_ln(e, efg[...], efb[...]))
    etab_out[...] = jnp.concatenate(
        [e, jnp.zeros((_L, 128 - _DE), _f32)], axis=1)


def _feat_call(node3, residf, edgemat, pa):
    in_specs = [
        pl.BlockSpec((1, _L, 6), lambda i: (i, 0, 0)),
        pl.BlockSpec((1, _L, 1), lambda i: (i, 0, 0)),
        _full_spec(edgemat),
    ] + [_full_spec(a) for a in pa]
    out_shape = [
        jax.ShapeDtypeStruct((_B, _L, _DN), _f32),
        jax.ShapeDtypeStruct((_B, _L, _DIDX), _f32),
        jax.ShapeDtypeStruct((_L, 128), _f32),
    ]
    out_specs = [
        pl.BlockSpec((1, _L, _DN), lambda i: (i, 0, 0)),
        pl.BlockSpec((1, _L, _DIDX), lambda i: (i, 0, 0)),
        pl.BlockSpec((_L, 128), lambda i: (0, 0)),
    ]
    return pl.pallas_call(
        _feat_body, grid=(_B,), in_specs=in_specs, out_specs=out_specs,
        out_shape=out_shape)(node3, residf, edgemat, *pa)


def _rgc_body(dn, dpad, final, *refs):
    node_ref, nbr_ref, edge_ref = refs[:3]
    npar = _N_RGC + (2 if final else 0)
    prefs = refs[3:3 + npar]
    outs = refs[3 + npar:]
    v = [r[...] for r in prefs]
    (eg1i, eb1i, eg1n, eb1n, eg1e, eb1e, W1i, W1n, W1e, bb1,
     eg2, eb2, W2, bb2, Wsci, Wscn, Wsce, bsc) = v[:18]
    rb2 = v[18:26]
    elng, elnb = v[26:28]
    n1 = v[28:38]
    n2 = v[38:46]
    nlng, nlnb = v[46:48]

    dcat = 2 * dn + _DE
    node = node_ref[...][:, :dn]
    nbr3 = nbr_ref[...][:, :, :dn]
    edge3 = edge_ref[...]
    nbr = nbr3.reshape(_K * _T, dn)
    edge = edge3.reshape(_K * _T, _DE)

    sc3 = (_dot(node, Wsci)[None] + _dot(nbr, Wscn).reshape(_K, _T, _DE)
           + _dot(edge, Wsce).reshape(_K, _T, _DE) + bsc[None])

    c = 1.0 / dcat
    s_i = jnp.sum(node, axis=1, keepdims=True)[None]
    ss_i = jnp.sum(node * node, axis=1, keepdims=True)[None]
    s = (s_i + jnp.sum(nbr3, axis=2, keepdims=True)
         + jnp.sum(edge3, axis=2, keepdims=True))
    ss = (ss_i + jnp.sum(nbr3 * nbr3, axis=2, keepdims=True)
          + jnp.sum(edge3 * edge3, axis=2, keepdims=True))
    mu, inv = _stats(s, ss, c)

    a_i = jax.nn.relu((node[None] - mu) * inv * eg1i[None] + eb1i[None])
    a_n = jax.nn.relu((nbr3 - mu) * inv * eg1n[None] + eb1n[None])
    a_e = jax.nn.relu((edge3 - mu) * inv * eg1e[None] + eb1e[None])
    h = (_dot(a_i.reshape(_K * _T, dn), W1i)
         + _dot(a_n.reshape(_K * _T, dn), W1n)
         + _dot(a_e.reshape(_K * _T, _DE), W1e) + bb1)
    h = jax.nn.relu(_ln(h, eg2, eb2))
    h = _dot(h, W2) + bb2
    r1 = h.reshape(_K, _T, _DE) + sc3
    r2 = _rb(r1.reshape(_K * _T, _DE), rb2)
    ef = jax.nn.relu(_ln(r2, elng, elnb))
    edge_out = edge3 + ef.reshape(_K, _T, _DE)

    m = jnp.sum(edge_out, axis=0) * (1.0 / _K)

    hn = jnp.concatenate([node, m], axis=1)
    hh = _rb(hn, n1)
    hh = _rb(hh, n2)
    node_out = node + jax.nn.relu(_ln(hh, nlng, nlnb))

    if final:
        Wout, bout = v[48:50]
        outs[0][...] = _dot(node_out, Wout) + bout
    else:
        if dpad > dn:
            node_out = jnp.concatenate(
                [node_out, jnp.zeros((_T, dpad - dn), _f32)], axis=1)
        outs[0][...] = node_out
        outs[1][...] = edge_out


def _rgc_call(node, nbr3, edge3, pv, dn, final, dpad=None):
    dpad = dn if dpad is None else dpad
    rows = node.shape[0]
    grid = (rows // _T,)
    in_specs = [
        pl.BlockSpec((_T, dpad), lambda i: (i, 0)),
        pl.BlockSpec((_K, _T, dpad), lambda i: (0, i, 0)),
        pl.BlockSpec((_K, _T, _DE), lambda i: (0, i, 0)),
    ] + [_full_spec(a) for a in pv]
    if final:
        out_shape = [jax.ShapeDtypeStruct((rows, _DOUT), _f32)]
        out_specs = [pl.BlockSpec((_T, _DOUT), lambda i: (i, 0))]
    else:
        out_shape = [jax.ShapeDtypeStruct((rows, dpad), _f32),
                     jax.ShapeDtypeStruct((_K, rows, _DE), _f32)]
        out_specs = [pl.BlockSpec((_T, dpad), lambda i: (i, 0)),
                     pl.BlockSpec((_K, _T, _DE), lambda i: (0, i, 0))]
    return pl.pallas_call(
        functools.partial(_rgc_body, dn, dpad, final),
        grid=grid, in_specs=in_specs, out_specs=out_specs,
        out_shape=out_shape)(node, nbr3, edge3, *pv)


def kernel(node_in, edgemat_in, adjmat_in, masked_resid, params):
    p = params

    adj = adjmat_in.astype(jnp.int32)
    off = (jnp.arange(_B, dtype=jnp.int32) * _L)[:, None, None]
    idxT3 = jnp.transpose(adj + off, (2, 0, 1)).reshape(_K, _BL)
    idxE = jnp.transpose(adj, (2, 0, 1)).reshape(1, _N)
    _H = _BL // 2
    idxH = [idxT3[:, :_H].reshape(1, -1), idxT3[:, _H:].reshape(1, -1)]

    residf = masked_resid.astype(_f32).reshape(_B, _L, 1)
    emb_pad = jnp.zeros((32, _DIDX), _f32).at[:21].set(p['idx_emb'])

    pa = [p['conv0']['W'].reshape(9 * 6, _DN), p['conv0']['b'][None, :]]
    pa += _frb(p['nf_blocks'][0]) + _frb(p['nf_blocks'][1])
    pa += _fl(p['nf_ln'])
    pa += [emb_pad] + _frb(p['idx_blocks'][0]) + _fl(p['idx_ln'])
    pa += _frb(p['ef_blocks'][0]) + _frb(p['ef_blocks'][1])
    pa += _fl(p['ef_ln'])

    node0, idxf, etab = _feat_call(node_in, residf, edgemat_in, pa)

    edge = _sc_gather(etab, idxE).reshape(_K, _BL, 128)[:, :, :_DE]
    node = node0.reshape(_BL, _DN)
    edge_h = [edge[:, :_H], edge[:, _H:]]

    for blk in p['encoder']:
        pv = _flat_rgc(blk, _DN)
        nbr_h = [_sc_gather(node, ix).reshape(_K, _H, _DN) for ix in idxH]
        res = [_rgc_call(node[c * _H:(c + 1) * _H], nbr_h[c], edge_h[c],
                         pv, _DN, final=False) for c in range(2)]
        node = jnp.concatenate([res[0][0], res[1][0]], axis=0)
        edge_h = [res[0][1], res[1][1]]

    _DP = 256
    node = jnp.concatenate(
        [node, idxf.reshape(_BL, _DIDX),
         jnp.zeros((_BL, _DP - _DND), _f32)], axis=1)

    for bi, blk in enumerate(p['decoder']):
        pv = _flat_rgc(blk, _DND)
        nbr_h = [_sc_gather(node, ix).reshape(_K, _H, _DP) for ix in idxH]
        if bi < len(p['decoder']) - 1:
            res = [_rgc_call(node[c * _H:(c + 1) * _H], nbr_h[c],
                             edge_h[c], pv, _DND, final=False, dpad=_DP)
                   for c in range(2)]
            node = jnp.concatenate([res[0][0], res[1][0]], axis=0)
            edge_h = [res[0][1], res[1][1]]
        else:
            pv += [p['out']['W'], p['out']['b'][None, :]]
            outs = [_rgc_call(node[c * _H:(c + 1) * _H], nbr_h[c],
                              edge_h[c], pv, _DND, final=True, dpad=_DP)[0]
                    for c in range(2)]
            out = jnp.concatenate(outs, axis=0)

    return out.reshape(_B, _L, _DOUT)
